# baseline (speedup 1.0000x reference)
import numpy as np

import jax
import jax.numpy as jnp
from jax.experimental import pallas as pl
from jax.experimental.pallas import tpu as pltpu

K = 8
R = 3
NT = 16
RADIUS = 40.0
N = 512


def _basis_np(k=K):
    th = 2.0 * np.pi * np.arange(k) / k
    return np.stack([np.cos(th), np.sin(th)], axis=-1).astype(np.float32)


def _shift_mats_np(k=K):
    s = np.arange(k)
    tgt = (np.arange(k)[None, :] + s[:, None]) % k
    out = np.zeros((k, k, k), dtype=np.float32)
    for si in range(k):
        for ki in range(k):
            out[si, ki, tgt[si, ki]] = 1.0
    return out


def _wshift(Wc):
    Sall = jnp.asarray(_shift_mats_np())
    Wsh = jnp.einsum('roc,skm->rsckom', Wc, Sall)
    Rr, Ss, Cc, Kk, Oo, Mm = Wsh.shape
    return Wsh.reshape(Rr * Ss, Cc * Kk, Oo * Mm)


def _wdmat(Wd):
    km = (np.arange(K)[None, :] - np.arange(K)[:, None]) % K
    W = Wd[:, :, jnp.asarray(km)]
    W = jnp.transpose(W, (1, 2, 0, 3))
    O = Wd.shape[0]
    C = Wd.shape[1]
    return W.reshape(C * K, O * K)


def _pool(C):
    return jnp.asarray(np.kron(np.eye(C, dtype=np.float32),
                               np.ones((K, K), dtype=np.float32)))


def _net_kernel(pos_row_ref, pos_col_ref, mask_ref, xin_ref,
                wproj_ref, wsh0_ref, wdm0_ref, wsh1_ref, wdm1_ref,
                wsh2_ref, wdm2_ref, wsh3_ref, wdm3_ref, wsh4_ref, wdm4_ref,
                pool64_ref, pool128_ref, out_ref, P_ref, Y_ref):
    f32 = jnp.float32
    px_row = pos_row_ref[0, 0:1, :]
    py_row = pos_row_ref[0, 1:2, :]
    px_col = pos_col_ref[0, :, 0:1]
    py_col = pos_col_ref[0, :, 1:2]
    dx = px_row - px_col
    dy = py_row - py_col
    d = jnp.sqrt(dx * dx + dy * dy + 1e-9)
    rn = jnp.minimum(d * (1.0 / RADIUS), 1.0)
    win = 1.0 - rn * rn
    win = win * win * win
    mask_row = mask_ref[0]
    winm = win * mask_row
    rpos = rn * (R - 1.0)
    ang = jnp.arctan2(dy, dx)
    tb = jnp.floor((ang + np.pi) * (NT / (2.0 * np.pi))).astype(jnp.int32)
    sb = jnp.bitwise_and(tb, K - 1)

    for r in range(R):
        ar = jnp.maximum(1.0 - jnp.abs(rpos - float(r)), 0.0) * winm
        for s in range(K):
            t = r * K + s
            P_ref[:, t * N:(t + 1) * N] = jnp.where(sb == s, ar, 0.0).astype(jnp.bfloat16)

    denom = jnp.sum(mask_ref[0])
    inv_denom = 1.0 / (denom + 1e-6)

    def matmul(a, b):
        return jax.lax.dot_general(a, b, (((1,), (0,)), ((), ())),
                                   preferred_element_type=f32)

    def cts(X, wsh_ref, OK):
        Xb = X.astype(jnp.bfloat16)
        for t in range(R * K):
            Y_ref[t * N:(t + 1) * N, :OK] = matmul(Xb, wsh_ref[t]).astype(jnp.bfloat16)
        return matmul(P_ref[...], Y_ref[:, :OK]) * inv_denom

    def nonlin(v, pool_ref):
        sq = v * v
        mags = matmul(sq, pool_ref[...]) + 1e-6
        return v * (jnp.maximum(mags - 0.2, 0.0) / mags)

    X0 = matmul(xin_ref[0], wproj_ref[...])

    oc = cts(X0, wsh0_ref, 4 * K)
    od = matmul(X0, wdm0_ref[...])
    prev = jnp.concatenate([oc, od], axis=1)

    X = nonlin(prev, pool64_ref)
    prev = cts(X, wsh1_ref, 8 * K) + matmul(X, wdm1_ref[...]) + prev

    X = nonlin(prev, pool64_ref)
    prev = cts(X, wsh2_ref, 16 * K) + matmul(X, wdm2_ref[...])

    X = nonlin(prev, pool128_ref)
    prev = cts(X, wsh3_ref, 8 * K) + matmul(X, wdm3_ref[...])

    X = nonlin(prev, pool64_ref)
    out_ref[0] = cts(X, wsh4_ref, 3 * K) + matmul(X, wdm4_ref[...])


def kernel(p0_enc, v0_enc, p0, v0, a, fluid_mask,
           Wc0, Wd0, Wc1, Wd1, Wc2, Wd2, Wc3, Wd3, Wc4, Wd4):
    f32 = jnp.float32
    Bm = p0.shape[0]
    dt = 1.0
    v1 = v0 + dt * a
    p1 = p0 + dt * (v0 + v1) / 2.0

    fluid_feats = jnp.concatenate(
        [v1[..., None, :], p1[..., None, :], v0_enc, p0_enc], axis=-2)
    xin = fluid_feats.reshape(Bm, N, 32)

    B8 = jnp.asarray(_basis_np())
    wproj = jnp.kron(jnp.eye(16, dtype=f32), B8.T)

    wsh = [_wshift(Wc0).astype(jnp.bfloat16), _wshift(Wc1).astype(jnp.bfloat16),
           _wshift(Wc2).astype(jnp.bfloat16), _wshift(Wc3).astype(jnp.bfloat16),
           _wshift(Wc4).astype(jnp.bfloat16)]
    wdm = [_wdmat(Wd0), _wdmat(Wd1), _wdmat(Wd2), _wdmat(Wd3), _wdmat(Wd4)]
    pool64 = _pool(8)
    pool128 = _pool(16)

    pos_row = jnp.transpose(p1, (0, 2, 1))
    maskr = fluid_mask[:, None, :]

    def rep(arr):
        nd = arr.ndim
        return pl.BlockSpec(arr.shape, lambda b: (0,) * nd)

    in_arrays = [pos_row, p1, maskr, xin,
                 wproj, wsh[0], wdm[0], wsh[1], wdm[1],
                 wsh[2], wdm[2], wsh[3], wdm[3], wsh[4], wdm[4],
                 pool64, pool128]
    in_specs = [
        pl.BlockSpec((1, 2, N), lambda b: (b, 0, 0)),
        pl.BlockSpec((1, N, 2), lambda b: (b, 0, 0)),
        pl.BlockSpec((1, 1, N), lambda b: (b, 0, 0)),
        pl.BlockSpec((1, N, 32), lambda b: (b, 0, 0)),
    ] + [rep(w) for w in in_arrays[4:]]

    out_reg = pl.pallas_call(
        _net_kernel,
        grid=(Bm,),
        in_specs=in_specs,
        out_specs=pl.BlockSpec((1, N, 3 * K), lambda b: (b, 0, 0)),
        out_shape=jax.ShapeDtypeStruct((Bm, N, 3 * K), f32),
        scratch_shapes=[pltpu.VMEM((N, R * K * N), jnp.bfloat16),
                        pltpu.VMEM((R * K * N, 128), jnp.bfloat16)],
    )(*in_arrays)

    out3 = out_reg.reshape(Bm, N, 3, K)
    pos_correction = (1.0 / 128.0) * (2.0 / K) * jnp.einsum('bnck,kd->bncd', out3, B8)
    p_corrected = p1 + pos_correction[..., 0, :]
    v_corrected = (p_corrected - p0) / dt
    m_matrix = pos_correction[..., 1:, :]
    return p_corrected, v_corrected, m_matrix, (v0_enc, p0_enc)


# speedup vs baseline: 1.3646x; 1.3646x over previous
import numpy as np

import jax
import jax.numpy as jnp
from jax.experimental import pallas as pl
from jax.experimental.pallas import tpu as pltpu

K = 8
R = 3
NT = 16
RADIUS = 40.0
N = 512
NB = 2
LAYER_CH = [(16, 4), (8, 8), (8, 16), (16, 8), (8, 3)]


def _basis_np(k=K):
    th = 2.0 * np.pi * np.arange(k) / k
    return np.stack([np.cos(th), np.sin(th)], axis=-1).astype(np.float32)


def _f32(x):
    return x.astype(jnp.float32)


def _iota(shape, dim):
    return jax.lax.broadcasted_iota(jnp.int32, shape, dim)


def _mm(a, b):
    return jax.lax.dot_general(a, b, (((1,), (0,)), ((), ())),
                               preferred_element_type=jnp.float32)


def _mm_t(a, b, adim, bdim):
    return jax.lax.dot_general(a, b, (((adim,), (bdim,)), ((), ())),
                               preferred_element_type=jnp.float32)


def _net_kernel(pos_row_ref, p1_ref, p0_ref, mask_ref, xin_ref,
                wproj_ref, pool64_ref, pool128_ref, qfin_ref,
                wc0_ref, wd0_ref, wc1_ref, wd1_ref, wc2_ref, wd2_ref,
                wc3_ref, wd3_ref, wc4_ref, wd4_ref,
                p_out_ref, v_out_ref, mm_out_ref,
                P_ref, wsh_refs, wdm_refs):
    f32 = jnp.float32
    bf16 = jnp.bfloat16
    wc_refs = [wc0_ref, wc1_ref, wc2_ref, wc3_ref, wc4_ref]
    wd_refs = [wd0_ref, wd1_ref, wd2_ref, wd3_ref, wd4_ref]

    for l, (C, O) in enumerate(LAYER_CH):
        CK, OK = C * K, O * K
        e1 = (_iota((CK, C), 0) // K == _iota((CK, C), 1)).astype(f32)
        e2 = (_iota((OK, O), 0) // K == _iota((OK, O), 1)).astype(f32)
        smod = (_iota((CK, OK), 1) % K - _iota((CK, OK), 0) % K) % K
        wc = wc_refs[l][...]
        for r in range(R):
            tmp = _mm_t(wc[r], e1, 1, 1)
            aexp = _mm_t(tmp, e2, 0, 1)
            for s in range(K):
                wsh_refs[l][r * K + s] = jnp.where(smod == s, aexp, 0.0).astype(bf16)
        wdraw = wd_refs[l][...]
        acc = jnp.zeros((CK, OK), dtype=f32)
        for t in range(K):
            sel = ((_iota((CK, CK), 1) // K == _iota((CK, CK), 0) // K)
                   & (_iota((CK, CK), 1) % K == t)).astype(f32)
            tmp = _mm_t(wdraw, sel, 1, 1)
            exp = _mm_t(tmp, e2, 0, 1)
            acc = acc + jnp.where(smod == t, exp, 0.0)
        wdm_refs[l][...] = acc

    def cts(Xb, l, OK, b):
        acc = jnp.zeros((N, OK), dtype=f32)
        for t in range(R * K):
            Yt = _mm(Xb, wsh_refs[l][t])
            acc = acc + _mm(P_ref[b, t], Yt.astype(bf16))
        return acc

    def nonlin(v, pool_ref):
        sq = v * v
        mags = _mm(sq, pool_ref[...]) + 1e-6
        return v * (jnp.maximum(mags - 0.2, 0.0) / mags)

    for b in range(NB):
        px_row = pos_row_ref[b, 0:1, :]
        py_row = pos_row_ref[b, 1:2, :]
        px_col = p1_ref[b, :, 0:1]
        py_col = p1_ref[b, :, 1:2]
        dx = px_row - px_col
        dy = py_row - py_col
        d = jnp.sqrt(dx * dx + dy * dy + 1e-9)
        rn = jnp.minimum(d * (1.0 / RADIUS), 1.0)
        win = 1.0 - rn * rn
        win = win * win * win
        mask_row = mask_ref[b]
        winm = win * mask_row
        rpos = rn * (R - 1.0)
        ang = jnp.arctan2(dy, dx)
        tb = jnp.floor((ang + np.pi) * (NT / (2.0 * np.pi))).astype(jnp.int32)
        sb = jnp.bitwise_and(tb, K - 1)

        for r in range(R):
            ar = jnp.maximum(1.0 - jnp.abs(rpos - float(r)), 0.0) * winm
            for s in range(K):
                P_ref[b, r * K + s] = jnp.where(sb == s, ar, 0.0).astype(bf16)

        denom = jnp.sum(mask_ref[b])
        inv_denom = 1.0 / (denom + 1e-6)

        X0 = _mm(xin_ref[b], wproj_ref[...])

        Xb = X0.astype(bf16)
        oc = cts(Xb, 0, 4 * K, b) * inv_denom
        od = _mm(X0, wdm_refs[0][...])
        prev = jnp.concatenate([oc, od], axis=1)

        X = nonlin(prev, pool64_ref)
        prev = cts(X.astype(bf16), 1, 8 * K, b) * inv_denom \
            + _mm(X, wdm_refs[1][...]) + prev

        X = nonlin(prev, pool64_ref)
        prev = cts(X.astype(bf16), 2, 16 * K, b) * inv_denom \
            + _mm(X, wdm_refs[2][...])

        X = nonlin(prev, pool128_ref)
        prev = cts(X.astype(bf16), 3, 8 * K, b) * inv_denom \
            + _mm(X, wdm_refs[3][...])

        X = nonlin(prev, pool64_ref)
        out24 = cts(X.astype(bf16), 4, 3 * K, b) * inv_denom \
            + _mm(X, wdm_refs[4][...])

        out6 = _mm(out24, qfin_ref[...])
        pc = p1_ref[b] + out6[:, 0:2]
        p_out_ref[b] = pc
        v_out_ref[b] = pc - p0_ref[b]
        mm_out_ref[b] = out6[:, 2:6]


def kernel(p0_enc, v0_enc, p0, v0, a, fluid_mask,
           Wc0, Wd0, Wc1, Wd1, Wc2, Wd2, Wc3, Wd3, Wc4, Wd4):
    f32 = jnp.float32
    Bm = p0.shape[0]
    dt = 1.0
    v1 = v0 + dt * a
    p1 = p0 + dt * (v0 + v1) / 2.0

    xin = jnp.concatenate(
        [v1, p1, v0_enc.reshape(Bm, N, 14), p0_enc.reshape(Bm, N, 14)],
        axis=-1)
    pos_row = jnp.transpose(p1, (0, 2, 1))
    maskr = fluid_mask[:, None, :]

    B8 = _basis_np()
    wproj = jnp.asarray(np.kron(np.eye(16, dtype=np.float32), B8.T))
    pool64 = jnp.asarray(np.kron(np.eye(8, dtype=np.float32),
                                 np.ones((K, K), dtype=np.float32)))
    pool128 = jnp.asarray(np.kron(np.eye(16, dtype=np.float32),
                                  np.ones((K, K), dtype=np.float32)))
    qfin = jnp.asarray((1.0 / 128.0) * (2.0 / K) *
                       np.kron(np.eye(3, dtype=np.float32), B8))

    wds = [Wd0.reshape(4, 16 * K), Wd1.reshape(8, 8 * K),
           Wd2.reshape(16, 8 * K), Wd3.reshape(8, 16 * K),
           Wd4.reshape(3, 8 * K)]

    in_arrays = [pos_row, p1, p0, maskr, xin,
                 wproj, pool64, pool128, qfin,
                 Wc0, wds[0], Wc1, wds[1], Wc2, wds[2],
                 Wc3, wds[3], Wc4, wds[4]]

    def full(arr):
        nd = arr.ndim
        return pl.BlockSpec(arr.shape, lambda *_: (0,) * nd)

    wsh_scratch = [pltpu.VMEM((R * K, C * K, O * K), jnp.bfloat16)
                   for (C, O) in LAYER_CH]
    wdm_scratch = [pltpu.VMEM((C * K, O * K), f32) for (C, O) in LAYER_CH]

    p_out, v_out, mm_out = pl.pallas_call(
        _net_kernel,
        grid=(1,),
        in_specs=[full(x) for x in in_arrays],
        out_specs=[full(jax.ShapeDtypeStruct((Bm, N, 2), f32)),
                   full(jax.ShapeDtypeStruct((Bm, N, 2), f32)),
                   full(jax.ShapeDtypeStruct((Bm, N, 4), f32))],
        out_shape=[jax.ShapeDtypeStruct((Bm, N, 2), f32),
                   jax.ShapeDtypeStruct((Bm, N, 2), f32),
                   jax.ShapeDtypeStruct((Bm, N, 4), f32)],
        scratch_shapes=[pltpu.VMEM((NB, R * K, N, N), jnp.bfloat16),
                        wsh_scratch, wdm_scratch],
    )(*in_arrays)

    m_matrix = mm_out.reshape(Bm, N, 2, 2)
    return p_out, v_out, m_matrix, (v0_enc, p0_enc)


# speedup vs baseline: 1.9365x; 1.4191x over previous
import numpy as np

import jax
import jax.numpy as jnp
from jax.experimental import pallas as pl
from jax.experimental.pallas import tpu as pltpu

K = 8
R = 3
NT = 16
RADIUS = 40.0
N = 512
NB = 2
LAYER_CH = [(16, 4), (8, 8), (8, 16), (16, 8), (8, 3)]


def _basis_np(k=K):
    th = 2.0 * np.pi * np.arange(k) / k
    return np.stack([np.cos(th), np.sin(th)], axis=-1).astype(np.float32)


def _f32(x):
    return x.astype(jnp.float32)


def _iota(shape, dim):
    return jax.lax.broadcasted_iota(jnp.int32, shape, dim)


def _mm(a, b):
    return jax.lax.dot_general(a, b, (((1,), (0,)), ((), ())),
                               preferred_element_type=jnp.float32)


def _mm_t(a, b, adim, bdim):
    return jax.lax.dot_general(a, b, (((adim,), (bdim,)), ((), ())),
                               preferred_element_type=jnp.float32)


def _net_kernel(pos_row_ref, p1_ref, p0_ref, mask_ref, xin_ref,
                wproj_ref, pool64_ref, pool128_ref, qfin_ref,
                wc0_ref, wd0_ref, wc1_ref, wd1_ref, wc2_ref, wd2_ref,
                wc3_ref, wd3_ref, wc4_ref, wd4_ref,
                p_out_ref, v_out_ref, mm_out_ref,
                P_ref, wsh_refs, wdm_refs):
    f32 = jnp.float32
    bf16 = jnp.bfloat16
    wc_refs = [wc0_ref, wc1_ref, wc2_ref, wc3_ref, wc4_ref]
    wd_refs = [wd0_ref, wd1_ref, wd2_ref, wd3_ref, wd4_ref]

    for l, (C, O) in enumerate(LAYER_CH):
        CK, OK = C * K, O * K
        e1 = (_iota((CK, C), 0) // K == _iota((CK, C), 1)).astype(f32)
        e2 = (_iota((OK, O), 0) // K == _iota((OK, O), 1)).astype(f32)
        smod = (_iota((CK, OK), 1) % K - _iota((CK, OK), 0) % K) % K
        wc = wc_refs[l][...]
        for r in range(R):
            tmp = _mm_t(wc[r], e1, 1, 1)
            aexp = _mm_t(tmp, e2, 0, 1)
            for s in range(K):
                t = r * K + s
                wsh_refs[l][:, t * OK:(t + 1) * OK] = \
                    jnp.where(smod == s, aexp, 0.0).astype(bf16)
        wdraw = wd_refs[l][...]
        acc = jnp.zeros((CK, OK), dtype=f32)
        for t in range(K):
            sel = ((_iota((CK, CK), 1) // K == _iota((CK, CK), 0) // K)
                   & (_iota((CK, CK), 1) % K == t)).astype(f32)
            tmp = _mm_t(wdraw, sel, 1, 1)
            exp = _mm_t(tmp, e2, 0, 1)
            acc = acc + jnp.where(smod == t, exp, 0.0)
        wdm_refs[l][...] = acc

    def cts(X, l, OK, b):
        Yall = _mm(X, wsh_refs[l][...]).astype(bf16)
        acc = jnp.zeros((N, OK), dtype=f32)
        for t in range(R * K):
            acc = acc + _mm(P_ref[b, t], Yall[:, t * OK:(t + 1) * OK])
        return acc

    def nonlin(v, pool_ref):
        sq = v * v
        mags = _mm(sq, pool_ref[...]) + 1e-6
        return v * (jnp.maximum(mags - 0.2, 0.0) / mags)

    inv_denom = []
    for b in range(NB):
        px_row = pos_row_ref[b, 0:1, :]
        py_row = pos_row_ref[b, 1:2, :]
        px_col = p1_ref[b, :, 0:1]
        py_col = p1_ref[b, :, 1:2]
        dx = px_row - px_col
        dy = py_row - py_col
        d = jnp.sqrt(dx * dx + dy * dy + 1e-9)
        rn = jnp.minimum(d * (1.0 / RADIUS), 1.0)
        win = 1.0 - rn * rn
        win = win * win * win
        mask_row = mask_ref[b]
        winm = win * mask_row
        rpos = rn * (R - 1.0)
        ang = jnp.arctan2(dy, dx)
        tb = jnp.floor((ang + np.pi) * (NT / (2.0 * np.pi))).astype(jnp.int32)
        sb = jnp.bitwise_and(tb, K - 1)

        for r in range(R):
            ar = jnp.maximum(1.0 - jnp.abs(rpos - float(r)), 0.0) * winm
            for s in range(K):
                P_ref[b, r * K + s] = jnp.where(sb == s, ar, 0.0).astype(bf16)

        inv_denom.append(1.0 / (jnp.sum(mask_ref[b]) + 1e-6))

    X0 = [_mm(xin_ref[b], wproj_ref[...]) for b in range(NB)]
    prev = [None, None]
    for b in range(NB):
        oc = cts(X0[b].astype(bf16), 0, 4 * K, b) * inv_denom[b]
        od = _mm(X0[b], wdm_refs[0][...])
        prev[b] = jnp.concatenate([oc, od], axis=1)

    for l, pool_r, res in ((1, pool64_ref, True), (2, pool64_ref, False),
                           (3, pool128_ref, False)):
        OK = LAYER_CH[l][1] * K
        for b in range(NB):
            X = nonlin(prev[b], pool_r)
            new = cts(X.astype(bf16), l, OK, b) * inv_denom[b] \
                + _mm(X, wdm_refs[l][...])
            prev[b] = new + prev[b] if res else new

    for b in range(NB):
        X = nonlin(prev[b], pool64_ref)
        out24 = cts(X.astype(bf16), 4, 3 * K, b) * inv_denom[b] \
            + _mm(X, wdm_refs[4][...])

        out6 = _mm(out24, qfin_ref[...])
        pc = p1_ref[b] + out6[:, 0:2]
        p_out_ref[b] = pc
        v_out_ref[b] = pc - p0_ref[b]
        mm_out_ref[b] = out6[:, 2:6]


def kernel(p0_enc, v0_enc, p0, v0, a, fluid_mask,
           Wc0, Wd0, Wc1, Wd1, Wc2, Wd2, Wc3, Wd3, Wc4, Wd4):
    f32 = jnp.float32
    Bm = p0.shape[0]
    dt = 1.0
    v1 = v0 + dt * a
    p1 = p0 + dt * (v0 + v1) / 2.0

    xin = jnp.concatenate(
        [v1, p1, v0_enc.reshape(Bm, N, 14), p0_enc.reshape(Bm, N, 14)],
        axis=-1)
    pos_row = jnp.transpose(p1, (0, 2, 1))
    maskr = fluid_mask[:, None, :]

    B8 = _basis_np()
    wproj = jnp.asarray(np.kron(np.eye(16, dtype=np.float32), B8.T))
    pool64 = jnp.asarray(np.kron(np.eye(8, dtype=np.float32),
                                 np.ones((K, K), dtype=np.float32)))
    pool128 = jnp.asarray(np.kron(np.eye(16, dtype=np.float32),
                                  np.ones((K, K), dtype=np.float32)))
    qfin = jnp.asarray((1.0 / 128.0) * (2.0 / K) *
                       np.kron(np.eye(3, dtype=np.float32), B8))

    wds = [Wd0.reshape(4, 16 * K), Wd1.reshape(8, 8 * K),
           Wd2.reshape(16, 8 * K), Wd3.reshape(8, 16 * K),
           Wd4.reshape(3, 8 * K)]

    in_arrays = [pos_row, p1, p0, maskr, xin,
                 wproj, pool64, pool128, qfin,
                 Wc0, wds[0], Wc1, wds[1], Wc2, wds[2],
                 Wc3, wds[3], Wc4, wds[4]]

    def full(arr):
        nd = arr.ndim
        return pl.BlockSpec(arr.shape, lambda *_: (0,) * nd)

    wsh_scratch = [pltpu.VMEM((C * K, R * K * O * K), jnp.bfloat16)
                   for (C, O) in LAYER_CH]
    wdm_scratch = [pltpu.VMEM((C * K, O * K), f32) for (C, O) in LAYER_CH]

    p_out, v_out, mm_out = pl.pallas_call(
        _net_kernel,
        grid=(1,),
        in_specs=[full(x) for x in in_arrays],
        out_specs=[full(jax.ShapeDtypeStruct((Bm, N, 2), f32)),
                   full(jax.ShapeDtypeStruct((Bm, N, 2), f32)),
                   full(jax.ShapeDtypeStruct((Bm, N, 4), f32))],
        out_shape=[jax.ShapeDtypeStruct((Bm, N, 2), f32),
                   jax.ShapeDtypeStruct((Bm, N, 2), f32),
                   jax.ShapeDtypeStruct((Bm, N, 4), f32)],
        scratch_shapes=[pltpu.VMEM((NB, R * K, N, N), jnp.bfloat16),
                        wsh_scratch, wdm_scratch],
    )(*in_arrays)

    m_matrix = mm_out.reshape(Bm, N, 2, 2)
    return p_out, v_out, m_matrix, (v0_enc, p0_enc)


# speedup vs baseline: 2.1237x; 1.0967x over previous
import numpy as np

import jax
import jax.numpy as jnp
from jax.experimental import pallas as pl
from jax.experimental.pallas import tpu as pltpu

K = 8
R = 3
NT = 16
RADIUS = 40.0
N = 512
NB = 2
LAYER_CH = [(16, 4), (8, 8), (8, 16), (16, 8), (8, 3)]


def _basis_np(k=K):
    th = 2.0 * np.pi * np.arange(k) / k
    return np.stack([np.cos(th), np.sin(th)], axis=-1).astype(np.float32)


def _f32(x):
    return x.astype(jnp.float32)


def _iota(shape, dim):
    return jax.lax.broadcasted_iota(jnp.int32, shape, dim)


def _mm(a, b):
    return jax.lax.dot_general(a, b, (((1,), (0,)), ((), ())),
                               preferred_element_type=jnp.float32)


def _mm_t(a, b, adim, bdim):
    return jax.lax.dot_general(a, b, (((adim,), (bdim,)), ((), ())),
                               preferred_element_type=jnp.float32)


def _net_kernel(pos_row_ref, p1_ref, p0_ref, mask_ref, xin_ref,
                wproj_ref, pool64_ref, pool128_ref, qfin_ref,
                wc0_ref, wd0_ref, wc1_ref, wd1_ref, wc2_ref, wd2_ref,
                wc3_ref, wd3_ref, wc4_ref, wd4_ref,
                p_out_ref, v_out_ref, mm_out_ref,
                P_ref, wsh_refs, wdm_refs):
    f32 = jnp.float32
    bf16 = jnp.bfloat16
    wc_refs = [wc0_ref, wc1_ref, wc2_ref, wc3_ref, wc4_ref]
    wd_refs = [wd0_ref, wd1_ref, wd2_ref, wd3_ref, wd4_ref]

    inv_denom = []
    for b in range(NB):
        px_row = pos_row_ref[b, 0:1, :]
        py_row = pos_row_ref[b, 1:2, :]
        px_col = p1_ref[b, :, 0:1]
        py_col = p1_ref[b, :, 1:2]
        dx = px_row - px_col
        dy = py_row - py_col
        d = jnp.sqrt(dx * dx + dy * dy + 1e-9)
        rn = jnp.minimum(d * (1.0 / RADIUS), 1.0)
        win = 1.0 - rn * rn
        win = win * win * win
        mask_row = mask_ref[b]
        winm = win * mask_row
        rpos = rn * (R - 1.0)
        u = dx * dx - dy * dy
        v = 2.0 * dx * dy
        A = (v < 0.0) | ((v == 0.0) & (u < 0.0))
        u2 = jnp.where(A, -u, u)
        v2 = jnp.where(A, -v, v)
        B = (u2 < 0.0) | ((u2 == 0.0) & (v2 > 0.0))
        u3 = jnp.where(B, v2, u2)
        v3 = jnp.where(B, -u2, v2)
        Cb = v3 > u3
        sb = (A.astype(jnp.int32) * 4 + B.astype(jnp.int32) * 2
              + Cb.astype(jnp.int32))
        smask = [(sb == s).astype(bf16) for s in range(K)]

        for r in range(R):
            ar = (jnp.maximum(1.0 - jnp.abs(rpos - float(r)), 0.0)
                  * winm).astype(bf16)
            for s in range(K):
                P_ref[b, r * K + s] = smask[s] * ar

        inv_denom.append(1.0 / (jnp.sum(mask_ref[b]) + 1e-6))

    for l, (C, O) in enumerate(LAYER_CH):
        CK, OK = C * K, O * K
        e1 = (_iota((CK, C), 0) // K == _iota((CK, C), 1)).astype(f32)
        e2 = (_iota((OK, O), 0) // K == _iota((OK, O), 1)).astype(f32)
        smod = (_iota((CK, OK), 1) % K - _iota((CK, OK), 0) % K) % K
        tmp = _mm_t(e1, wc_refs[l][...], 1, 1)
        i0 = _iota((R * OK, R * O), 0)
        i1 = _iota((R * OK, R * O), 1)
        e2r = ((i0 // OK == i1 // O) & ((i0 % OK) // K == i1 % O)).astype(f32)
        aexp_all = _mm_t(tmp, e2r, 1, 1)
        for r in range(R):
            aexp = aexp_all[:, r * OK:(r + 1) * OK]
            for s in range(K):
                t = r * K + s
                wsh_refs[l][:, t * OK:(t + 1) * OK] = \
                    jnp.where(smod == s, aexp, 0.0).astype(bf16)
        T = _mm_t(e2, wd_refs[l][...], 1, 0)
        jmod = _iota((OK, CK), 1) % K
        Tbig = jnp.concatenate([T * (jmod == t).astype(f32) for t in range(K)],
                               axis=0)
        e3 = (_iota((CK, CK), 1) // K == _iota((CK, CK), 0) // K).astype(f32)
        expbig = _mm_t(e3, Tbig, 1, 1)
        acc = jnp.zeros((CK, OK), dtype=f32)
        for t in range(K):
            acc = acc + jnp.where(smod == t,
                                  expbig[:, t * OK:(t + 1) * OK], 0.0)
        wdm_refs[l][...] = acc

    def cts(X, l, OK, b):
        Yall = _mm(X, wsh_refs[l][...]).astype(bf16)
        acc = jnp.zeros((N, OK), dtype=f32)
        for t in range(R * K):
            acc = acc + _mm(P_ref[b, t], Yall[:, t * OK:(t + 1) * OK])
        return acc

    def nonlin(v, pool_ref):
        sq = v * v
        mags = _mm(sq, pool_ref[...]) + 1e-6
        return v * (jnp.maximum(mags - 0.2, 0.0) / mags)

    X0 = [_mm(xin_ref[b], wproj_ref[...]) for b in range(NB)]
    prev = [None, None]
    for b in range(NB):
        oc = cts(X0[b].astype(bf16), 0, 4 * K, b) * inv_denom[b]
        od = _mm(X0[b], wdm_refs[0][...])
        prev[b] = jnp.concatenate([oc, od], axis=1)

    for l, pool_r, res in ((1, pool64_ref, True), (2, pool64_ref, False),
                           (3, pool128_ref, False)):
        OK = LAYER_CH[l][1] * K
        for b in range(NB):
            X = nonlin(prev[b], pool_r)
            new = cts(X.astype(bf16), l, OK, b) * inv_denom[b] \
                + _mm(X, wdm_refs[l][...])
            prev[b] = new + prev[b] if res else new

    for b in range(NB):
        X = nonlin(prev[b], pool64_ref)
        out24 = cts(X.astype(bf16), 4, 3 * K, b) * inv_denom[b] \
            + _mm(X, wdm_refs[4][...])

        out6 = _mm(out24, qfin_ref[...])
        pc = p1_ref[b] + out6[:, 0:2]
        p_out_ref[b] = pc
        v_out_ref[b] = pc - p0_ref[b]
        mm_out_ref[b] = out6[:, 2:6]


def kernel(p0_enc, v0_enc, p0, v0, a, fluid_mask,
           Wc0, Wd0, Wc1, Wd1, Wc2, Wd2, Wc3, Wd3, Wc4, Wd4):
    f32 = jnp.float32
    Bm = p0.shape[0]
    dt = 1.0
    v1 = v0 + dt * a
    p1 = p0 + dt * (v0 + v1) / 2.0

    xin = jnp.concatenate(
        [v1, p1, v0_enc.reshape(Bm, N, 14), p0_enc.reshape(Bm, N, 14)],
        axis=-1)
    pos_row = jnp.transpose(p1, (0, 2, 1))
    maskr = fluid_mask[:, None, :]

    B8 = _basis_np()
    wproj = jnp.asarray(np.kron(np.eye(16, dtype=np.float32), B8.T))
    pool64 = jnp.asarray(np.kron(np.eye(8, dtype=np.float32),
                                 np.ones((K, K), dtype=np.float32)))
    pool128 = jnp.asarray(np.kron(np.eye(16, dtype=np.float32),
                                  np.ones((K, K), dtype=np.float32)))
    qfin = jnp.asarray((1.0 / 128.0) * (2.0 / K) *
                       np.kron(np.eye(3, dtype=np.float32), B8))

    wds = [Wd0.reshape(4, 16 * K), Wd1.reshape(8, 8 * K),
           Wd2.reshape(16, 8 * K), Wd3.reshape(8, 16 * K),
           Wd4.reshape(3, 8 * K)]

    wcs = [Wc0.reshape(R * 4, 16), Wc1.reshape(R * 8, 8),
           Wc2.reshape(R * 16, 8), Wc3.reshape(R * 8, 16),
           Wc4.reshape(R * 3, 8)]

    in_arrays = [pos_row, p1, p0, maskr, xin,
                 wproj, pool64, pool128, qfin,
                 wcs[0], wds[0], wcs[1], wds[1], wcs[2], wds[2],
                 wcs[3], wds[3], wcs[4], wds[4]]

    def full(arr):
        nd = arr.ndim
        return pl.BlockSpec(arr.shape, lambda *_: (0,) * nd)

    wsh_scratch = [pltpu.VMEM((C * K, R * K * O * K), jnp.bfloat16)
                   for (C, O) in LAYER_CH]
    wdm_scratch = [pltpu.VMEM((C * K, O * K), f32) for (C, O) in LAYER_CH]

    p_out, v_out, mm_out = pl.pallas_call(
        _net_kernel,
        grid=(1,),
        in_specs=[full(x) for x in in_arrays],
        out_specs=[full(jax.ShapeDtypeStruct((Bm, N, 2), f32)),
                   full(jax.ShapeDtypeStruct((Bm, N, 2), f32)),
                   full(jax.ShapeDtypeStruct((Bm, N, 4), f32))],
        out_shape=[jax.ShapeDtypeStruct((Bm, N, 2), f32),
                   jax.ShapeDtypeStruct((Bm, N, 2), f32),
                   jax.ShapeDtypeStruct((Bm, N, 4), f32)],
        scratch_shapes=[pltpu.VMEM((NB, R * K, N, N), jnp.bfloat16),
                        wsh_scratch, wdm_scratch],
    )(*in_arrays)

    m_matrix = mm_out.reshape(Bm, N, 2, 2)
    return p_out, v_out, m_matrix, (v0_enc, p0_enc)


# speedup vs baseline: 2.2785x; 1.0729x over previous
import numpy as np

import jax
import jax.numpy as jnp
from jax.experimental import pallas as pl
from jax.experimental.pallas import tpu as pltpu

K = 8
R = 3
NT = 16
RADIUS = 40.0
N = 512
NB = 2
LAYER_CH = [(16, 4), (8, 8), (8, 16), (16, 8), (8, 3)]


def _basis_np(k=K):
    th = 2.0 * np.pi * np.arange(k) / k
    return np.stack([np.cos(th), np.sin(th)], axis=-1).astype(np.float32)


def _f32(x):
    return x.astype(jnp.float32)


def _iota(shape, dim):
    return jax.lax.broadcasted_iota(jnp.int32, shape, dim)


def _mm(a, b):
    return jax.lax.dot_general(a, b, (((1,), (0,)), ((), ())),
                               preferred_element_type=jnp.float32)


def _mm_t(a, b, adim, bdim):
    return jax.lax.dot_general(a, b, (((adim,), (bdim,)), ((), ())),
                               preferred_element_type=jnp.float32)


def _net_kernel(pos_row_ref, p1_ref, p0_ref, mask_ref, xin_ref,
                wproj_ref, pool64_ref, pool128_ref, qfin_ref,
                wc0_ref, wd0_ref, wc1_ref, wd1_ref, wc2_ref, wd2_ref,
                wc3_ref, wd3_ref, wc4_ref, wd4_ref,
                p_out_ref, v_out_ref, mm_out_ref,
                P_ref, wsh_refs, wdm_refs, acc_ref):
    f32 = jnp.float32
    bf16 = jnp.bfloat16
    wc_refs = [wc0_ref, wc1_ref, wc2_ref, wc3_ref, wc4_ref]
    wd_refs = [wd0_ref, wd1_ref, wd2_ref, wd3_ref, wd4_ref]

    inv_denom = []
    far_flags = []
    for b in range(NB):
        px_row = pos_row_ref[b, 0:1, :]
        py_row = pos_row_ref[b, 1:2, :]
        px_col = p1_ref[b, :, 0:1]
        py_col = p1_ref[b, :, 1:2]
        dx = px_row - px_col
        dy = py_row - py_col
        d = jnp.sqrt(dx * dx + dy * dy + 1e-9)
        rn = jnp.minimum(d * (1.0 / RADIUS), 1.0)
        win = 1.0 - rn * rn
        win = win * win * win
        mask_row = mask_ref[b]
        winm = win * mask_row
        rpos = rn * (R - 1.0)
        u = dx * dx - dy * dy
        v = 2.0 * dx * dy
        A = (v < 0.0) | ((v == 0.0) & (u < 0.0))
        u2 = jnp.where(A, -u, u)
        v2 = jnp.where(A, -v, v)
        B = (u2 < 0.0) | ((u2 == 0.0) & (v2 > 0.0))
        u3 = jnp.where(B, v2, u2)
        v3 = jnp.where(B, -u2, v2)
        Cb = v3 > u3
        sb = (A.astype(jnp.int32) * 4 + B.astype(jnp.int32) * 2
              + Cb.astype(jnp.int32))
        smask = [(sb == s).astype(bf16) for s in range(K)]

        for r in range(2):
            ar = (jnp.maximum(1.0 - jnp.abs(rpos - float(r)), 0.0)
                  * winm).astype(bf16)
            for s in range(K):
                P_ref[b, r * K + s] = smask[s] * ar

        far_flags.append(jnp.max(rpos) > 1.0)

        @pl.when(far_flags[b])
        def _():
            ar = (jnp.maximum(1.0 - jnp.abs(rpos - 2.0), 0.0)
                  * winm).astype(bf16)
            for s in range(K):
                P_ref[b, 2 * K + s] = smask[s] * ar

        inv_denom.append(1.0 / (jnp.sum(mask_ref[b]) + 1e-6))
    far = far_flags[0] | far_flags[1]

    for l, (C, O) in enumerate(LAYER_CH):
        CK, OK = C * K, O * K
        e1 = (_iota((CK, C), 0) // K == _iota((CK, C), 1)).astype(f32)
        e2 = (_iota((OK, O), 0) // K == _iota((OK, O), 1)).astype(f32)
        smod = (_iota((CK, OK), 1) % K - _iota((CK, OK), 0) % K) % K
        tmp = _mm_t(e1, wc_refs[l][...], 1, 1)
        i0 = _iota((R * OK, R * O), 0)
        i1 = _iota((R * OK, R * O), 1)
        e2r = ((i0 // OK == i1 // O) & ((i0 % OK) // K == i1 % O)).astype(f32)
        aexp_all = _mm_t(tmp, e2r, 1, 1)
        for r in range(R):
            aexp = aexp_all[:, r * OK:(r + 1) * OK]
            for s in range(K):
                t = r * K + s
                wsh_refs[l][:, t * OK:(t + 1) * OK] = \
                    jnp.where(smod == s, aexp, 0.0).astype(bf16)
        T = _mm_t(e2, wd_refs[l][...], 1, 0)
        jmod = _iota((OK, CK), 1) % K
        Tbig = jnp.concatenate([T * (jmod == t).astype(f32) for t in range(K)],
                               axis=0)
        e3 = (_iota((CK, CK), 1) // K == _iota((CK, CK), 0) // K).astype(f32)
        expbig = _mm_t(e3, Tbig, 1, 1)
        acc = jnp.zeros((CK, OK), dtype=f32)
        for t in range(K):
            acc = acc + jnp.where(smod == t,
                                  expbig[:, t * OK:(t + 1) * OK], 0.0)
        wdm_refs[l][...] = acc

    def cts(X, l, OK, b):
        Yall = _mm(X, wsh_refs[l][:, :16 * OK]).astype(bf16)
        acc = jnp.zeros((N, OK), dtype=f32)
        for t in range(16):
            acc = acc + _mm(P_ref[b, t], Yall[:, t * OK:(t + 1) * OK])
        acc_ref[b, :, :OK] = acc

        @pl.when(far)
        def _():
            Y2 = _mm(X, wsh_refs[l][:, 16 * OK:]).astype(bf16)
            acc2 = jnp.zeros((N, OK), dtype=f32)
            for t in range(16, R * K):
                acc2 = acc2 + _mm(P_ref[b, t],
                                  Y2[:, (t - 16) * OK:(t - 15) * OK])
            acc_ref[b, :, :OK] += acc2

        return acc_ref[b, :, :OK]

    def nonlin(v, pool_ref):
        sq = v * v
        mags = _mm(sq, pool_ref[...]) + 1e-6
        return v * (jnp.maximum(mags - 0.2, 0.0) / mags)

    X0 = [_mm(xin_ref[b], wproj_ref[...]) for b in range(NB)]
    prev = [None, None]
    for b in range(NB):
        oc = cts(X0[b].astype(bf16), 0, 4 * K, b) * inv_denom[b]
        od = _mm(X0[b], wdm_refs[0][...])
        prev[b] = jnp.concatenate([oc, od], axis=1)

    for l, pool_r, res in ((1, pool64_ref, True), (2, pool64_ref, False),
                           (3, pool128_ref, False)):
        OK = LAYER_CH[l][1] * K
        for b in range(NB):
            X = nonlin(prev[b], pool_r)
            new = cts(X.astype(bf16), l, OK, b) * inv_denom[b] \
                + _mm(X, wdm_refs[l][...])
            prev[b] = new + prev[b] if res else new

    for b in range(NB):
        X = nonlin(prev[b], pool64_ref)
        out24 = cts(X.astype(bf16), 4, 3 * K, b) * inv_denom[b] \
            + _mm(X, wdm_refs[4][...])

        out6 = _mm(out24, qfin_ref[...])
        pc = p1_ref[b] + out6[:, 0:2]
        p_out_ref[b] = pc
        v_out_ref[b] = pc - p0_ref[b]
        mm_out_ref[b] = out6[:, 2:6]


def kernel(p0_enc, v0_enc, p0, v0, a, fluid_mask,
           Wc0, Wd0, Wc1, Wd1, Wc2, Wd2, Wc3, Wd3, Wc4, Wd4):
    f32 = jnp.float32
    Bm = p0.shape[0]
    dt = 1.0
    v1 = v0 + dt * a
    p1 = p0 + dt * (v0 + v1) / 2.0

    xin = jnp.concatenate(
        [v1, p1, v0_enc.reshape(Bm, N, 14), p0_enc.reshape(Bm, N, 14)],
        axis=-1)
    pos_row = jnp.transpose(p1, (0, 2, 1))
    maskr = fluid_mask[:, None, :]

    B8 = _basis_np()
    wproj = jnp.asarray(np.kron(np.eye(16, dtype=np.float32), B8.T))
    pool64 = jnp.asarray(np.kron(np.eye(8, dtype=np.float32),
                                 np.ones((K, K), dtype=np.float32)))
    pool128 = jnp.asarray(np.kron(np.eye(16, dtype=np.float32),
                                  np.ones((K, K), dtype=np.float32)))
    qfin = jnp.asarray((1.0 / 128.0) * (2.0 / K) *
                       np.kron(np.eye(3, dtype=np.float32), B8))

    wds = [Wd0.reshape(4, 16 * K), Wd1.reshape(8, 8 * K),
           Wd2.reshape(16, 8 * K), Wd3.reshape(8, 16 * K),
           Wd4.reshape(3, 8 * K)]

    wcs = [Wc0.reshape(R * 4, 16), Wc1.reshape(R * 8, 8),
           Wc2.reshape(R * 16, 8), Wc3.reshape(R * 8, 16),
           Wc4.reshape(R * 3, 8)]

    in_arrays = [pos_row, p1, p0, maskr, xin,
                 wproj, pool64, pool128, qfin,
                 wcs[0], wds[0], wcs[1], wds[1], wcs[2], wds[2],
                 wcs[3], wds[3], wcs[4], wds[4]]

    def full(arr):
        nd = arr.ndim
        return pl.BlockSpec(arr.shape, lambda *_: (0,) * nd)

    wsh_scratch = [pltpu.VMEM((C * K, R * K * O * K), jnp.bfloat16)
                   for (C, O) in LAYER_CH]
    wdm_scratch = [pltpu.VMEM((C * K, O * K), f32) for (C, O) in LAYER_CH]

    p_out, v_out, mm_out = pl.pallas_call(
        _net_kernel,
        grid=(1,),
        in_specs=[full(x) for x in in_arrays],
        out_specs=[full(jax.ShapeDtypeStruct((Bm, N, 2), f32)),
                   full(jax.ShapeDtypeStruct((Bm, N, 2), f32)),
                   full(jax.ShapeDtypeStruct((Bm, N, 4), f32))],
        out_shape=[jax.ShapeDtypeStruct((Bm, N, 2), f32),
                   jax.ShapeDtypeStruct((Bm, N, 2), f32),
                   jax.ShapeDtypeStruct((Bm, N, 4), f32)],
        scratch_shapes=[pltpu.VMEM((NB, R * K, N, N), jnp.bfloat16),
                        wsh_scratch, wdm_scratch,
                        pltpu.VMEM((NB, N, 128), f32)],
    )(*in_arrays)

    m_matrix = mm_out.reshape(Bm, N, 2, 2)
    return p_out, v_out, m_matrix, (v0_enc, p0_enc)
